# baseline (speedup 1.0000x reference)
import dataclasses
import functools

import jax
import jax.numpy as jnp
from jax import lax
from jax.experimental import pallas as pl
from jax.experimental.pallas import tpu as pltpu
from jax.experimental.pallas import tpu_sc as plsc

N = 10000
E = 160000
D = 256
HALF = 128
ALPHA_C = 0.1

NC = 2
NS = 16

ROWS = E // 128
SLAB = 40
ZCH = 400
NCH = N // ZCH


def _mesh():
    return plsc.VectorSubcoreMesh(core_axis_name="c", subcore_axis_name="s")


def _no_layout_passes():
    cp = pltpu.CompilerParams()
    if "needs_layout_passes" in pltpu.CompilerParams.__dataclass_fields__:
        cp = dataclasses.replace(cp, needs_layout_passes=False)
    return cp


DEG_ROWS_PER_W = 40
DEG_COLS = 640


def _prep_sc(x0, x1, dstp_pad):
    @functools.partial(
        pl.kernel,
        out_type=(jax.ShapeDtypeStruct((N, HALF), jnp.float32),
                  jax.ShapeDtypeStruct((N, HALF), jnp.float32),
                  jax.ShapeDtypeStruct((NC * N,), jnp.float32)),
        mesh=_mesh(),
        compiler_params=_no_layout_passes(),
        scratch_types=[
            pltpu.VMEM_SHARED((NS * N,), jnp.float32),
            pltpu.VMEM((N,), jnp.float32),
            pltpu.VMEM((80, 128), jnp.int32),
            pltpu.VMEM((NS * DEG_COLS,), jnp.float32),
            pltpu.VMEM((DEG_COLS,), jnp.float32),
            pltpu.VMEM((160, HALF), jnp.float32),
        ],
    )
    def k(x0_hbm, x1_hbm, dstp_hbm, y0_hbm, y1_hbm, dis_hbm,
          stage, hist_v, idx_v, part_v, dis_v, xb):
        c = lax.axis_index("c")
        s = lax.axis_index("s")

        @pl.loop(0, N // 16)
        def _(i):
            hist_v[pl.ds(i * 16, 16)] = jnp.zeros((16,), jnp.float32)

        base = 80 * s
        pltpu.sync_copy(dstp_hbm.at[pl.ds(base, 80)], idx_v)
        ones16 = jnp.full((16,), 1.0, jnp.float32)

        @pl.loop(0, 80)
        def _(r):
            @pl.when(base + r < ROWS)
            def _():
                for g in range(8):
                    idx16 = idx_v[r, pl.ds(g * 16, 16)]
                    plsc.addupdate_scatter(hist_v, [idx16], ones16)

        pltpu.sync_copy(hist_v, stage.at[pl.ds(s * N, N)])
        plsc.subcore_barrier()

        a = 624 * s
        for p in range(NS):
            pltpu.sync_copy(stage.at[pl.ds(p * N + a, DEG_COLS)],
                            part_v.at[pl.ds(p * DEG_COLS, DEG_COLS)])

        @pl.loop(0, DEG_COLS // 16)
        def _(ch):
            acc16 = part_v[pl.ds(ch * 16, 16)]
            for p in range(1, NS):
                acc16 = acc16 + part_v[pl.ds(p * DEG_COLS + ch * 16, 16)]
            d = acc16 + 1.0
            bits = plsc.bitcast(d, jnp.int32)
            bits = jnp.int32(0x5F3759DF) - (bits >> 1)
            r0 = plsc.bitcast(bits, jnp.float32)
            for _ in range(3):
                r0 = r0 * (1.5 - 0.5 * d * r0 * r0)
            dis_v[pl.ds(ch * 16, 16)] = r0

        pltpu.sync_copy(dis_v, dis_hbm.at[pl.ds(c * N + a, DEG_COLS)])

        def scale(x_hbm, y_hbm):
            for b in range(4):
                rb = a + 160 * b
                pltpu.sync_copy(x_hbm.at[pl.ds(rb, 160)], xb)

                @pl.loop(0, 160)
                def _(r):
                    lr = 160 * b + r
                    dvec = plsc.load_gather(
                        dis_v, [jnp.zeros((16,), jnp.int32) + lr])
                    for g in range(8):
                        sl = pl.ds(g * 16, 16)
                        xb[r, sl] = xb[r, sl] * dvec

                pltpu.sync_copy(xb, y_hbm.at[pl.ds(rb, 160)])

        @pl.when(c == 0)
        def _():
            scale(x0_hbm, y0_hbm)

        @pl.when(c == 1)
        def _():
            scale(x1_hbm, y1_hbm)

    return k(x0, x1, dstp_pad)


def _scatter_sc(srcp_pad, dstp_pad, y0, y1, zeros128):
    @functools.partial(
        pl.kernel,
        out_type=(jax.ShapeDtypeStruct((N, HALF), jnp.float32),
                  jax.ShapeDtypeStruct((N, HALF), jnp.float32)),
        mesh=_mesh(),
        scratch_types=[
            pltpu.VMEM_SHARED((N, HALF), jnp.float32),
            pltpu.VMEM((SLAB, 128), jnp.int32),
            pltpu.VMEM((SLAB, 128), jnp.int32),
            pltpu.VMEM((128, HALF), jnp.float32),
            pltpu.VMEM((128, HALF), jnp.float32),
            pltpu.SemaphoreType.DMA,
            pltpu.SemaphoreType.DMA,
        ],
    )
    def k(srcp_hbm, dstp_hbm, y0_hbm, y1_hbm, z_hbm, s0_hbm, s1_hbm,
          acc, slab_s, slab_d, rows0, rows1, sem0, sem1):
        c = lax.axis_index("c")
        s = lax.axis_index("s")

        rows = (rows0, rows1)
        sems = (sem0, sem1)

        def work(y_hbm):
            for h in range(2):
                base = SLAB * (2 * s + h)
                pltpu.sync_copy(srcp_hbm.at[pl.ds(base, SLAB)], slab_s)
                pltpu.sync_copy(dstp_hbm.at[pl.ds(base, SLAB)], slab_d)

                def start_gather(p, kk, base=base):
                    ok = jnp.logical_and(base + kk < ROWS, kk < SLAB)

                    @pl.when(ok)
                    def _():
                        pltpu.make_async_copy(
                            y_hbm.at[slab_s.at[kk]], rows[p], sems[p]).start()

                start_gather(0, 0)
                start_gather(1, 1)

                if h == 0:
                    @pl.loop(s, NCH, step=NS)
                    def _(i):
                        pltpu.sync_copy(z_hbm, acc.at[pl.ds(i * ZCH, ZCH)])

                    plsc.subcore_barrier()

                @pl.loop(0, SLAB // 2)
                def _(kk2):
                    for p in (0, 1):
                        kk = 2 * kk2 + p

                        @pl.when(base + kk < ROWS)
                        def _():
                            pltpu.make_async_copy(
                                y_hbm.at[slab_s.at[kk]], rows[p], sems[p]).wait()
                            pltpu.sync_copy(rows[p], acc.at[slab_d.at[kk]],
                                            add=True)
                            start_gather(p, kk + 2)

        @pl.when(c == 0)
        def _():
            work(y0_hbm)

        @pl.when(c == 1)
        def _():
            work(y1_hbm)

        plsc.subcore_barrier()

        def out_copy(o_hbm):
            @pl.loop(s, NCH, step=NS)
            def _(i):
                sl = pl.ds(i * ZCH, ZCH)
                pltpu.sync_copy(acc.at[sl], o_hbm.at[sl])

        @pl.when(c == 0)
        def _():
            out_copy(s0_hbm)

        @pl.when(c == 1)
        def _():
            out_copy(s1_hbm)

    return k(srcp_pad, dstp_pad, y0, y1, zeros128)


_R = 1000


def _finish_tc(x, s0, s1, dis2d):
    def body(x_ref, s0_ref, s1_ref, dis_ref, o_ref):
        dis = dis_ref[...]
        xv = x_ref[...]
        y = xv * dis
        sfull = jnp.concatenate([s0_ref[...], s1_ref[...]], axis=1)
        agg = dis * (sfull + y)
        h = (1.0 - ALPHA_C) * agg + ALPHA_C * xv
        o_ref[...] = xv + jnp.maximum(h, 0.0)

    return pl.pallas_call(
        body,
        grid=(N // _R,),
        in_specs=[
            pl.BlockSpec((_R, D), lambda i: (i, 0)),
            pl.BlockSpec((_R, HALF), lambda i: (i, 0)),
            pl.BlockSpec((_R, HALF), lambda i: (i, 0)),
            pl.BlockSpec((_R, 1), lambda i: (i, 0)),
        ],
        out_specs=pl.BlockSpec((_R, D), lambda i: (i, 0)),
        out_shape=jax.ShapeDtypeStruct((N, D), jnp.float32),
    )(x, s0, s1, dis2d)


@jax.jit
def kernel(x, edge_index):
    ei = edge_index.astype(jnp.int32)
    planes = jnp.pad(ei.reshape(2, ROWS, 128), ((0, 0), (0, 30), (0, 0)))
    srcp_pad = planes[0]
    dstp_pad = planes[1]

    zeros128 = jnp.zeros((ZCH, HALF), jnp.float32)

    y0, y1, disflat = _prep_sc(x[:, :HALF], x[:, HALF:], dstp_pad)
    s0, s1 = _scatter_sc(srcp_pad, dstp_pad, y0, y1, zeros128)
    return _finish_tc(x, s0, s1, disflat[:N].reshape(N, 1))


# speedup vs baseline: 1.0223x; 1.0223x over previous
import dataclasses
import functools

import jax
import jax.numpy as jnp
from jax import lax
from jax.experimental import pallas as pl
from jax.experimental.pallas import tpu as pltpu
from jax.experimental.pallas import tpu_sc as plsc

N = 10000
E = 160000
D = 256
HALF = 128
ALPHA_C = 0.1

NC = 2
NS = 16

ROWS = E // 128
SLAB = 40
ZCH = 400
NCH = N // ZCH


def _mesh():
    return plsc.VectorSubcoreMesh(core_axis_name="c", subcore_axis_name="s")


def _no_layout_passes():
    cp = pltpu.CompilerParams()
    if "needs_layout_passes" in pltpu.CompilerParams.__dataclass_fields__:
        cp = dataclasses.replace(cp, needs_layout_passes=False)
    return cp


DEG_ROWS_PER_W = 40
DEG_COLS = 640


def _deg_sc(dstp_pad):
    @functools.partial(
        pl.kernel,
        out_type=jax.ShapeDtypeStruct((NC * N,), jnp.float32),
        mesh=_mesh(),
        compiler_params=_no_layout_passes(),
        scratch_types=[
            pltpu.VMEM_SHARED((NS * N,), jnp.float32),
            pltpu.VMEM((N,), jnp.float32),
            pltpu.VMEM((DEG_ROWS_PER_W, 128), jnp.int32),
            pltpu.VMEM((NS * DEG_COLS,), jnp.float32),
            pltpu.VMEM((DEG_COLS,), jnp.float32),
        ],
    )
    def k(dstp_hbm, out_hbm, stage, hist_v, idx_v, part_v, out_v):
        c = lax.axis_index("c")
        s = lax.axis_index("s")
        w = s * NC + c

        @pl.loop(0, N // 16)
        def _(i):
            hist_v[pl.ds(i * 16, 16)] = jnp.zeros((16,), jnp.float32)

        start = DEG_ROWS_PER_W * w
        pltpu.sync_copy(dstp_hbm.at[pl.ds(start, DEG_ROWS_PER_W)], idx_v)

        ones16 = jnp.full((16,), 1.0, jnp.float32)

        @pl.loop(0, DEG_ROWS_PER_W)
        def _(r):
            @pl.when(start + r < ROWS)
            def _():
                for g in range(8):
                    idx16 = idx_v[r, pl.ds(g * 16, 16)]
                    plsc.addupdate_scatter(hist_v, [idx16], ones16)

        pltpu.sync_copy(hist_v, stage.at[pl.ds(s * N, N)])
        plsc.subcore_barrier()

        a = 624 * s
        for p in range(NS):
            pltpu.sync_copy(stage.at[pl.ds(p * N + a, DEG_COLS)],
                            part_v.at[pl.ds(p * DEG_COLS, DEG_COLS)])

        @pl.loop(0, DEG_COLS // 16)
        def _(ch):
            acc16 = part_v[pl.ds(ch * 16, 16)]
            for p in range(1, NS):
                acc16 = acc16 + part_v[pl.ds(p * DEG_COLS + ch * 16, 16)]
            out_v[pl.ds(ch * 16, 16)] = acc16

        pltpu.sync_copy(out_v, out_hbm.at[pl.ds(c * N + a, DEG_COLS)])

    return k(dstp_pad)


def _scatter_sc(srcp_pad, dstp_pad, y0, y1, zeros128):
    @functools.partial(
        pl.kernel,
        out_type=(jax.ShapeDtypeStruct((N, HALF), jnp.float32),
                  jax.ShapeDtypeStruct((N, HALF), jnp.float32)),
        mesh=_mesh(),
        scratch_types=[
            pltpu.VMEM_SHARED((N, HALF), jnp.float32),
            pltpu.VMEM((SLAB, 128), jnp.int32),
            pltpu.VMEM((SLAB, 128), jnp.int32),
            pltpu.VMEM((128, HALF), jnp.float32),
            pltpu.VMEM((128, HALF), jnp.float32),
            pltpu.SemaphoreType.DMA,
            pltpu.SemaphoreType.DMA,
        ],
    )
    def k(srcp_hbm, dstp_hbm, y0_hbm, y1_hbm, z_hbm, s0_hbm, s1_hbm,
          acc, slab_s, slab_d, rows0, rows1, sem0, sem1):
        c = lax.axis_index("c")
        s = lax.axis_index("s")

        rows = (rows0, rows1)
        sems = (sem0, sem1)

        def work(y_hbm):
            for h in range(2):
                base = SLAB * (2 * s + h)
                pltpu.sync_copy(srcp_hbm.at[pl.ds(base, SLAB)], slab_s)
                pltpu.sync_copy(dstp_hbm.at[pl.ds(base, SLAB)], slab_d)

                def start_gather(p, kk, base=base):
                    ok = jnp.logical_and(base + kk < ROWS, kk < SLAB)

                    @pl.when(ok)
                    def _():
                        pltpu.make_async_copy(
                            y_hbm.at[slab_s.at[kk]], rows[p], sems[p]).start()

                start_gather(0, 0)
                start_gather(1, 1)

                if h == 0:
                    @pl.loop(s, NCH, step=NS)
                    def _(i):
                        pltpu.sync_copy(z_hbm, acc.at[pl.ds(i * ZCH, ZCH)])

                    plsc.subcore_barrier()

                @pl.loop(0, SLAB // 2)
                def _(kk2):
                    for p in (0, 1):
                        kk = 2 * kk2 + p

                        @pl.when(base + kk < ROWS)
                        def _():
                            pltpu.make_async_copy(
                                y_hbm.at[slab_s.at[kk]], rows[p], sems[p]).wait()
                            pltpu.sync_copy(rows[p], acc.at[slab_d.at[kk]],
                                            add=True)
                            start_gather(p, kk + 2)

        @pl.when(c == 0)
        def _():
            work(y0_hbm)

        @pl.when(c == 1)
        def _():
            work(y1_hbm)

        plsc.subcore_barrier()

        def out_copy(o_hbm):
            @pl.loop(s, NCH, step=NS)
            def _(i):
                sl = pl.ds(i * ZCH, ZCH)
                pltpu.sync_copy(acc.at[sl], o_hbm.at[sl])

        @pl.when(c == 0)
        def _():
            out_copy(s0_hbm)

        @pl.when(c == 1)
        def _():
            out_copy(s1_hbm)

    return k(srcp_pad, dstp_pad, y0, y1, zeros128)


_R = 1000


def _scale_tc(x, d0, d1):
    def body(x_ref, d0_ref, d1_ref, y0_ref, y1_ref):
        deg = d0_ref[...] + d1_ref[...] + 1.0
        dis = lax.rsqrt(deg)
        y = x_ref[...] * dis
        y0_ref[...] = y[:, :HALF]
        y1_ref[...] = y[:, HALF:]

    return pl.pallas_call(
        body,
        grid=(N // _R,),
        in_specs=[
            pl.BlockSpec((_R, D), lambda i: (i, 0)),
            pl.BlockSpec((_R, 1), lambda i: (i, 0)),
            pl.BlockSpec((_R, 1), lambda i: (i, 0)),
        ],
        out_specs=[
            pl.BlockSpec((_R, HALF), lambda i: (i, 0)),
            pl.BlockSpec((_R, HALF), lambda i: (i, 0)),
        ],
        out_shape=[
            jax.ShapeDtypeStruct((N, HALF), jnp.float32),
            jax.ShapeDtypeStruct((N, HALF), jnp.float32),
        ],
    )(x, d0, d1)


def _finish_tc(x, s0, s1, d0, d1):
    def body(x_ref, s0_ref, s1_ref, d0_ref, d1_ref, o_ref):
        deg = d0_ref[...] + d1_ref[...] + 1.0
        dis = lax.rsqrt(deg)
        xv = x_ref[...]
        y = xv * dis
        sfull = jnp.concatenate([s0_ref[...], s1_ref[...]], axis=1)
        agg = dis * (sfull + y)
        h = (1.0 - ALPHA_C) * agg + ALPHA_C * xv
        o_ref[...] = xv + jnp.maximum(h, 0.0)

    return pl.pallas_call(
        body,
        grid=(N // _R,),
        in_specs=[
            pl.BlockSpec((_R, D), lambda i: (i, 0)),
            pl.BlockSpec((_R, HALF), lambda i: (i, 0)),
            pl.BlockSpec((_R, HALF), lambda i: (i, 0)),
            pl.BlockSpec((_R, 1), lambda i: (i, 0)),
            pl.BlockSpec((_R, 1), lambda i: (i, 0)),
        ],
        out_specs=pl.BlockSpec((_R, D), lambda i: (i, 0)),
        out_shape=jax.ShapeDtypeStruct((N, D), jnp.float32),
    )(x, s0, s1, d0, d1)


@jax.jit
def kernel(x, edge_index):
    ei = edge_index.astype(jnp.int32)
    planes = jnp.pad(ei.reshape(2, ROWS, 128), ((0, 0), (0, 30), (0, 0)))
    srcp_pad = planes[0]
    dstp_pad = planes[1]

    zeros128 = jnp.zeros((ZCH, HALF), jnp.float32)

    dflat = _deg_sc(dstp_pad)
    d0 = dflat[:N].reshape(N, 1)
    d1 = dflat[N:].reshape(N, 1)
    y0, y1 = _scale_tc(x, d0, d1)
    s0, s1 = _scatter_sc(srcp_pad, dstp_pad, y0, y1, zeros128)
    return _finish_tc(x, s0, s1, d0, d1)
